# Initial kernel scaffold; baseline (speedup 1.0000x reference)
import jax
import jax.numpy as jnp
from jax.experimental import pallas as pl


def kernel(x, W_in, W_k, W_v, W_q, W_alpha, d_alpha, b_alpha, W_out):
    raise NotImplementedError("write your pallas kernel here")


# speedup vs baseline: 3.7688x; 3.7688x over previous
import jax
import jax.numpy as jnp
from jax.experimental import pallas as pl
from jax.experimental.pallas import tpu as pltpu

DIM = 1024
NS = 64
PROJ_ROWS = 512
T_CHUNK = 128
B_BLK = 8


def _proj_kernel(x_ref, w_in_ref, w_cat_ref, bias_ref,
                 k_ref, v_ref, q_ref, ax_ref):
    xp = jnp.dot(x_ref[...], w_in_ref[...], preferred_element_type=jnp.float32)
    xp = xp * jax.nn.sigmoid(xp)
    kvqa = jnp.dot(xp, w_cat_ref[...], preferred_element_type=jnp.float32)
    kvqa = kvqa + bias_ref[...]
    k_ref[...] = kvqa[:, 0:NS]
    v_ref[...] = kvqa[:, NS:2 * NS]
    q_ref[...] = kvqa[:, 2 * NS:3 * NS]
    ax_ref[...] = kvqa[:, 3 * NS:4 * NS]


def _scan_kernel(k_ref, v_ref, q_ref, ax_ref, d_ref, wout_ref,
                 out_ref, sfin_ref, s_ref, o_ref):
    @pl.when(pl.program_id(1) == 0)
    def _():
        s_ref[...] = jnp.zeros_like(s_ref)

    d_row = d_ref[...]

    def body(tt, S):
        kt = k_ref[:, pl.ds(tt, 1), :].reshape(B_BLK, NS)
        vt = v_ref[:, pl.ds(tt, 1), :].reshape(B_BLK, NS)
        qt = q_ref[:, pl.ds(tt, 1), :].reshape(B_BLK, NS)
        axt = ax_ref[:, pl.ds(tt, 1), :].reshape(B_BLK, NS)
        retrieved = jnp.sum(S * kt[:, None, :], axis=-1)
        alpha = jax.nn.sigmoid(axt + d_row * retrieved)
        a = alpha[:, :, None]
        S = a * S + (1.0 - a) * (vt[:, :, None] * kt[:, None, :])
        o = jnp.sum(S * qt[:, None, :], axis=-1)
        o = o * o * jax.nn.sigmoid(o)
        o_ref[:, pl.ds(tt, 1), :] = o.reshape(B_BLK, 1, NS)
        return S

    S = jax.lax.fori_loop(0, T_CHUNK, body, s_ref[...])
    s_ref[...] = S
    sfin_ref[...] = S

    cell = o_ref[...].reshape(B_BLK * T_CHUNK, NS)
    out = jnp.dot(cell, wout_ref[...], preferred_element_type=jnp.float32)
    out_ref[...] = out.reshape(B_BLK, T_CHUNK, DIM)


def kernel(x, W_in, W_k, W_v, W_q, W_alpha, d_alpha, b_alpha, W_out):
    B, T, D = x.shape
    x2 = x.reshape(B * T, D)
    W_in_T = W_in.T
    W_cat = jnp.concatenate([W_k, W_v, W_q, W_alpha], axis=0).T
    bias = jnp.concatenate(
        [jnp.zeros((3 * NS,), jnp.float32), b_alpha])[None, :]

    n_rows = B * T
    grid_a = (n_rows // PROJ_ROWS,)
    pr_spec = pl.BlockSpec((PROJ_ROWS, NS), lambda i: (i, 0))
    k2, v2, q2, ax2 = pl.pallas_call(
        _proj_kernel,
        grid=grid_a,
        in_specs=[
            pl.BlockSpec((PROJ_ROWS, DIM), lambda i: (i, 0)),
            pl.BlockSpec((DIM, DIM), lambda i: (0, 0)),
            pl.BlockSpec((DIM, 4 * NS), lambda i: (0, 0)),
            pl.BlockSpec((1, 4 * NS), lambda i: (0, 0)),
        ],
        out_specs=[pr_spec, pr_spec, pr_spec, pr_spec],
        out_shape=[jax.ShapeDtypeStruct((n_rows, NS), jnp.float32)] * 4,
        compiler_params=pltpu.CompilerParams(
            dimension_semantics=("parallel",)),
    )(x2, W_in_T, W_cat, bias)

    shp = (B, T, NS)
    k3, v3, q3, ax3 = (a.reshape(shp) for a in (k2, v2, q2, ax2))

    grid_b = (B // B_BLK, T // T_CHUNK)
    in_spec = pl.BlockSpec((B_BLK, T_CHUNK, NS), lambda b, t: (b, t, 0))
    out, s_final = pl.pallas_call(
        _scan_kernel,
        grid=grid_b,
        in_specs=[
            in_spec, in_spec, in_spec, in_spec,
            pl.BlockSpec((1, NS), lambda b, t: (0, 0)),
            pl.BlockSpec((NS, DIM), lambda b, t: (0, 0)),
        ],
        out_specs=[
            pl.BlockSpec((B_BLK, T_CHUNK, DIM), lambda b, t: (b, t, 0)),
            pl.BlockSpec((B_BLK, NS, NS), lambda b, t: (b, 0, 0)),
        ],
        out_shape=[
            jax.ShapeDtypeStruct((B, T, DIM), jnp.float32),
            jax.ShapeDtypeStruct((B, NS, NS), jnp.float32),
        ],
        scratch_shapes=[
            pltpu.VMEM((B_BLK, NS, NS), jnp.float32),
            pltpu.VMEM((B_BLK, T_CHUNK, NS), jnp.float32),
        ],
        compiler_params=pltpu.CompilerParams(
            dimension_semantics=("parallel", "arbitrary")),
    )(k3, v3, q3, ax3, d_alpha[None, :], W_out.T)

    return out, s_final


# speedup vs baseline: 10.5972x; 2.8118x over previous
import jax
import jax.numpy as jnp
from jax.experimental import pallas as pl
from jax.experimental.pallas import tpu as pltpu

DIM = 1024
NS = 64
PROJ_ROWS = 512
B_BLK = 8
LW = B_BLK * NS
TC = 64


def _proj_kernel(x_ref, w_in_ref, w_cat_ref, bias_ref,
                 k_ref, v_ref, q_ref, ax_ref):
    xp = jnp.dot(x_ref[...], w_in_ref[...], preferred_element_type=jnp.float32)
    xp = xp * jax.nn.sigmoid(xp)
    kvqa = jnp.dot(xp, w_cat_ref[...], preferred_element_type=jnp.float32)
    kvqa = kvqa + bias_ref[...]
    k_ref[...] = kvqa[:, 0:NS]
    v_ref[...] = kvqa[:, NS:2 * NS]
    q_ref[...] = kvqa[:, 2 * NS:3 * NS]
    ax_ref[...] = kvqa[:, 3 * NS:4 * NS]


def _scan_kernel(kt_ref, qt_ref, v_ref, ax_ref, d_ref, e8_ref, wout_ref,
                 out_ref, sfin_ref, s_ref, kb_ref, qb_ref, o_ref):
    kb_ref[...] = jnp.dot(kt_ref[...].reshape(TC * NS, B_BLK), e8_ref[...],
                          preferred_element_type=jnp.float32)
    qb_ref[...] = jnp.dot(qt_ref[...].reshape(TC * NS, B_BLK), e8_ref[...],
                          preferred_element_type=jnp.float32)

    @pl.when(pl.program_id(1) == 0)
    def _():
        s_ref[...] = jnp.zeros_like(s_ref)

    d_row = d_ref[...]

    def body(tt, S):
        base = tt * NS
        kbt = kb_ref[pl.ds(base, NS), :]
        r = jnp.sum(S * kbt, axis=0, keepdims=True)
        z = ax_ref[pl.ds(tt, 1), :] + d_row * r
        alpha = jax.nn.sigmoid(z)
        w = (1.0 - alpha) * v_ref[pl.ds(tt, 1), :]
        S = alpha * S + w * kbt
        qbt = qb_ref[pl.ds(base, NS), :]
        o = jnp.sum(S * qbt, axis=0, keepdims=True)
        o = o * o * jax.nn.sigmoid(o)
        o_ref[pl.ds(tt, 1), :] = o
        return S

    S = jax.lax.fori_loop(0, TC, body, s_ref[...])
    s_ref[...] = S

    @pl.when(pl.program_id(1) == pl.num_programs(1) - 1)
    def _():
        for b in range(B_BLK):
            sfin_ref[b, :, :] = S[:, b * NS:(b + 1) * NS].T

    cell = jnp.concatenate(
        [o_ref[:, b * NS:(b + 1) * NS] for b in range(B_BLK)], axis=0)
    out = jnp.dot(cell, wout_ref[...], preferred_element_type=jnp.float32)
    out_ref[...] = out.reshape(B_BLK, TC, DIM)


def kernel(x, W_in, W_k, W_v, W_q, W_alpha, d_alpha, b_alpha, W_out):
    B, T, D = x.shape
    x2 = x.reshape(B * T, D)
    W_in_T = W_in.T
    W_cat = jnp.concatenate([W_k, W_v, W_q, W_alpha], axis=0).T
    bias = jnp.concatenate(
        [jnp.zeros((3 * NS,), jnp.float32), b_alpha])[None, :]

    n_rows = B * T
    grid_a = (n_rows // PROJ_ROWS,)
    pr_spec = pl.BlockSpec((PROJ_ROWS, NS), lambda i: (i, 0))
    k2, v2, q2, ax2 = pl.pallas_call(
        _proj_kernel,
        grid=grid_a,
        in_specs=[
            pl.BlockSpec((PROJ_ROWS, DIM), lambda i: (i, 0)),
            pl.BlockSpec((DIM, DIM), lambda i: (0, 0)),
            pl.BlockSpec((DIM, 4 * NS), lambda i: (0, 0)),
            pl.BlockSpec((1, 4 * NS), lambda i: (0, 0)),
        ],
        out_specs=[pr_spec, pr_spec, pr_spec, pr_spec],
        out_shape=[jax.ShapeDtypeStruct((n_rows, NS), jnp.float32)] * 4,
        compiler_params=pltpu.CompilerParams(
            dimension_semantics=("parallel",)),
    )(x2, W_in_T, W_cat, bias)

    n_half = B // B_BLK
    def to_tjb(a):
        return (a.reshape(n_half, B_BLK, T, NS)
                 .transpose(0, 2, 3, 1).reshape(n_half, T * NS, B_BLK))

    def to_rows(a):
        return (a.reshape(B, T, NS).transpose(1, 0, 2).reshape(T, B * NS))

    kt, qt = to_tjb(k2), to_tjb(q2)
    v_rows, ax_rows = to_rows(v2), to_rows(ax2)
    d_bi = jnp.tile(d_alpha, B)[None, :]
    e8 = jnp.repeat(jnp.eye(B_BLK, dtype=jnp.float32), NS, axis=1)

    grid_b = (n_half, T // TC)
    out, s_final = pl.pallas_call(
        _scan_kernel,
        grid=grid_b,
        in_specs=[
            pl.BlockSpec((1, TC * NS, B_BLK), lambda h, t: (h, t, 0)),
            pl.BlockSpec((1, TC * NS, B_BLK), lambda h, t: (h, t, 0)),
            pl.BlockSpec((TC, LW), lambda h, t: (t, h)),
            pl.BlockSpec((TC, LW), lambda h, t: (t, h)),
            pl.BlockSpec((1, LW), lambda h, t: (0, h)),
            pl.BlockSpec((B_BLK, LW), lambda h, t: (0, 0)),
            pl.BlockSpec((NS, DIM), lambda h, t: (0, 0)),
        ],
        out_specs=[
            pl.BlockSpec((B_BLK, TC, DIM), lambda h, t: (h, t, 0)),
            pl.BlockSpec((B_BLK, NS, NS), lambda h, t: (h, 0, 0)),
        ],
        out_shape=[
            jax.ShapeDtypeStruct((B, T, DIM), jnp.float32),
            jax.ShapeDtypeStruct((B, NS, NS), jnp.float32),
        ],
        scratch_shapes=[
            pltpu.VMEM((NS, LW), jnp.float32),
            pltpu.VMEM((TC * NS, LW), jnp.float32),
            pltpu.VMEM((TC * NS, LW), jnp.float32),
            pltpu.VMEM((TC, LW), jnp.float32),
        ],
        compiler_params=pltpu.CompilerParams(
            dimension_semantics=("parallel", "arbitrary")),
    )(kt, qt, v_rows, ax_rows, d_bi, e8, W_out.T)

    return out, s_final
